# Initial kernel scaffold; baseline (speedup 1.0000x reference)
import jax
import jax.numpy as jnp
from jax.experimental import pallas as pl


def kernel(segment, label_mapping):
    raise NotImplementedError("write your pallas kernel here")


# speedup vs baseline: 684.1562x; 684.1562x over previous
import functools

import jax
import jax.numpy as jnp
from jax import lax
from jax.experimental import pallas as pl
from jax.experimental.pallas import tpu as pltpu
from jax.experimental.pallas import tpu_sc as plsc

K = 150
KP = 160
MIN_RATIO = 0.02
NC = 2
NS = 16
NW = NC * NS
L = 16
NGROUPS = KP // L


def _hist_body(seg_hbm, part_hbm, chunk_v, bins_v, counts_v, *, words_per_w, chunk):
    c = lax.axis_index("c")
    s = lax.axis_index("s")
    wid = s * NC + c
    z = jnp.zeros((L,), jnp.int32)
    for r in range(NS):
        for g in range(NGROUPS):
            bins_v[r, pl.ds(g * L, L)] = z
    lane = lax.iota(jnp.int32, L)
    ones = jnp.ones((L,), jnp.int32)
    base = wid * words_per_w

    def chunk_iter(k, carry):
        pltpu.sync_copy(seg_hbm.at[pl.ds(base + k * chunk, chunk)], chunk_v)

        def vec_iter(i, carry2):
            v = chunk_v[pl.ds(i * L, L)]
            plsc.addupdate_scatter(bins_v, [lane, v], ones)
            return carry2

        lax.fori_loop(jnp.int32(0), jnp.int32(chunk // L), vec_iter, 0)
        return carry

    lax.fori_loop(jnp.int32(0), jnp.int32(words_per_w // chunk), chunk_iter, 0)

    for g in range(NGROUPS):
        acc = z
        for r in range(NS):
            acc = acc + bins_v[r, pl.ds(g * L, L)]
        counts_v[pl.ds(g * L, L)] = acc
    pltpu.sync_copy(counts_v, part_hbm.at[wid])


def _table_body(part_hbm, lm_hbm, table_hbm, part_v, lm_v, guide_v, counts_v,
                tab_v, *, min_pixels):
    c = lax.axis_index("c")
    s = lax.axis_index("s")
    wid = s * NC + c

    @pl.when(wid < NGROUPS)
    def _():
        g = wid
        pltpu.sync_copy(part_hbm, part_v)
        pltpu.sync_copy(lm_hbm, lm_v)
        z = jnp.zeros((L,), jnp.int32)
        for gg in range(NGROUPS):
            acc = z
            for w in range(NW):
                acc = acc + part_v[w, pl.ds(gg * L, L)]
            counts_v[pl.ds(gg * L, L)] = acc
            guide_v[pl.ds(gg * L, L)] = (acc >= min_pixels).astype(jnp.int32)
        cnt = counts_v[pl.ds(g * L, L)]
        small = (cnt > 0) & (cnt < min_pixels)
        cols = lax.iota(jnp.int32, L) + g * L
        matched = lm_v[K - 1, pl.ds(g * L, L)]
        for r in range(K - 1, -1, -1):
            cand = lm_v[r, pl.ds(g * L, L)]
            isg = plsc.load_gather(guide_v, [cand])
            matched = jnp.where(isg > 0, cand, matched)
        tab_v[:] = jnp.where(small, matched, cols)
        pltpu.sync_copy(tab_v, table_hbm.at[pl.ds(g * L, L)])


def _remap_body(seg_hbm, table_hbm, out_hbm, tab_v, in_v, out_v, *,
                words_per_w, chunk):
    c = lax.axis_index("c")
    s = lax.axis_index("s")
    wid = s * NC + c
    pltpu.sync_copy(table_hbm, tab_v)
    base = wid * words_per_w

    def chunk_iter(k, carry):
        off = base + k * chunk
        pltpu.sync_copy(seg_hbm.at[pl.ds(off, chunk)], in_v)

        def vec_iter(i, carry2):
            v = in_v[pl.ds(i * L, L)]
            out_v[pl.ds(i * L, L)] = plsc.load_gather(tab_v, [v])
            return carry2

        lax.fori_loop(jnp.int32(0), jnp.int32(chunk // L), vec_iter, 0)
        pltpu.sync_copy(out_v, out_hbm.at[pl.ds(off, chunk)])
        return carry

    lax.fori_loop(jnp.int32(0), jnp.int32(words_per_w // chunk), chunk_iter, 0)


@functools.partial(jax.jit, static_argnames=())
def kernel(segment, label_mapping):
    B, C, H, W = segment.shape
    n = B * C * H * W
    min_pixels = max(int(H * W * MIN_RATIO), 10)
    words_per_w = n // NW
    chunk = min(words_per_w, 16384)

    seg32 = segment.reshape(-1).astype(jnp.int32)
    lm32 = jnp.pad(label_mapping.astype(jnp.int32), ((0, 0), (0, KP - K)))

    mesh = plsc.VectorSubcoreMesh(
        core_axis_name="c", subcore_axis_name="s", num_cores=NC, num_subcores=NS)
    cparams = pltpu.CompilerParams(use_tc_tiling_on_sc=False, needs_layout_passes=False)

    hist = pl.kernel(
        functools.partial(_hist_body, words_per_w=words_per_w, chunk=chunk),
        out_type=jax.ShapeDtypeStruct((NW, KP), jnp.int32),
        mesh=mesh,
        scratch_types=[
            pltpu.VMEM((chunk,), jnp.int32),
            pltpu.VMEM((NS, KP), jnp.int32),
            pltpu.VMEM((KP,), jnp.int32),
        ],
        compiler_params=cparams,
    )
    partials = hist(seg32)

    table_k = pl.kernel(
        functools.partial(_table_body, min_pixels=min_pixels),
        out_type=jax.ShapeDtypeStruct((KP,), jnp.int32),
        mesh=mesh,
        scratch_types=[
            pltpu.VMEM((NW, KP), jnp.int32),
            pltpu.VMEM((K, KP), jnp.int32),
            pltpu.VMEM((KP,), jnp.int32),
            pltpu.VMEM((KP,), jnp.int32),
            pltpu.VMEM((L,), jnp.int32),
        ],
        compiler_params=cparams,
    )
    table = table_k(partials, lm32)

    remap = pl.kernel(
        functools.partial(_remap_body, words_per_w=words_per_w, chunk=chunk),
        out_type=jax.ShapeDtypeStruct((n,), jnp.int32),
        mesh=mesh,
        scratch_types=[
            pltpu.VMEM((KP,), jnp.int32),
            pltpu.VMEM((chunk,), jnp.int32),
            pltpu.VMEM((chunk,), jnp.int32),
        ],
        compiler_params=cparams,
    )
    out32 = remap(seg32, table)
    return out32.reshape(segment.shape).astype(segment.dtype)
